# baseline (speedup 1.0000x reference)
import functools

import jax
import jax.numpy as jnp
from jax import lax
from jax.experimental import pallas as pl
from jax.experimental.pallas import tpu as pltpu
from jax.experimental.pallas import tpu_sc as plsc

E = 8
TOPK = 2
D = 2048
FF = 512
FFS = 512
T = 2048
TILE_M = 128
NT = 40
ROWS = NT * TILE_M
NW = 32
TPW = T // NW
CHT = 16
NCH = TPW // CHT
CHC = 8
NCC = TPW // CHC


def _routing_body(x_ref, rw_ref, dst_ref, wb0_ref, wb1_ref, meta_ref, contrib_ref, cum_ref):
    x = x_ref[...]
    rw = rw_ref[...]
    logits = lax.dot_general(x, rw, (((1,), (1,)), ((), ())),
                             preferred_element_type=jnp.float32)
    m = jnp.max(logits, axis=1, keepdims=True)
    ex = jnp.exp(logits - m)
    probs = ex / jnp.sum(ex, axis=1, keepdims=True)

    iota8 = lax.broadcasted_iota(jnp.int32, (T, E), 1)
    v1 = jnp.max(probs, axis=1, keepdims=True)
    i1 = jnp.min(jnp.where(probs == v1, iota8, E), axis=1, keepdims=True)
    p2 = jnp.where(iota8 == i1, -1.0, probs)
    v2 = jnp.max(p2, axis=1, keepdims=True)
    i2 = jnp.min(jnp.where(p2 == v2, iota8, E), axis=1, keepdims=True)
    denom = v1 + v2
    w1 = v1 / denom
    w2 = v2 / denom

    oh1 = (iota8 == i1).astype(jnp.float32)
    oh2 = (iota8 == i2).astype(jnp.float32)
    contrib_ref[...] = oh1 + oh2

    CH = 256
    r = lax.broadcasted_iota(jnp.int32, (CH, CH), 0)
    c = lax.broadcasted_iota(jnp.int32, (CH, CH), 1)
    tri = (r > c).astype(jnp.float32)

    def step(ch, carry):
        blk = contrib_ref[pl.ds(ch * CH, CH), :]
        cum_blk = lax.dot_general(tri, blk, (((1,), (0,)), ((), ())),
                                  preferred_element_type=jnp.float32)
        cum_ref[pl.ds(ch * CH, CH), :] = cum_blk + carry
        return carry + jnp.sum(blk, axis=0, keepdims=True)

    counts = lax.fori_loop(0, T // CH, step, jnp.zeros((1, E), jnp.float32))

    ci = counts.astype(jnp.int32)
    ntiles = (ci + (TILE_M - 1)) // TILE_M
    e_r = lax.broadcasted_iota(jnp.int32, (E, E), 0)
    e_c = lax.broadcasted_iota(jnp.int32, (E, E), 1)
    mlt = (e_r < e_c).astype(jnp.float32)
    tile_off = lax.dot_general(ntiles.astype(jnp.float32), mlt,
                               (((1,), (0,)), ((), ())),
                               preferred_element_type=jnp.float32)
    row_off = tile_off * float(TILE_M)

    cum = cum_ref[...]
    sel1 = (iota8 == i1).astype(jnp.float32)
    sel2 = (iota8 == i2).astype(jnp.float32)
    dst0 = jnp.sum((cum + row_off) * sel1, axis=1, keepdims=True)
    dst1 = jnp.sum((cum + row_off) * sel2, axis=1, keepdims=True)

    colsel0 = (iota8 == 0).astype(jnp.float32)
    colsel1 = (iota8 == 1).astype(jnp.float32)
    dst_ref[...] = (dst0 * colsel0 + dst1 * colsel1).astype(jnp.int32)
    wb0_ref[...] = jnp.broadcast_to(w1, (T, 16))
    wb1_ref[...] = jnp.broadcast_to(w2, (T, 16))

    tile_end = (tile_off + ntiles.astype(jnp.float32))
    i8 = (e_r == e_c).astype(jnp.float32)
    ends_col = lax.dot_general(i8, tile_end, (((1,), (1,)), ((), ())),
                               preferred_element_type=jnp.float32)
    ends_b = jnp.broadcast_to(ends_col, (E, 128))
    jot = lax.broadcasted_iota(jnp.int32, (E, 128), 1).astype(jnp.float32)
    eid = jnp.sum((ends_b <= jot).astype(jnp.float32), axis=0, keepdims=True)
    eid = jnp.minimum(eid, float(E - 1))
    n_active = jnp.sum(ntiles.astype(jnp.float32), axis=1, keepdims=True)
    lane = lax.broadcasted_iota(jnp.int32, (1, 128), 1)
    meta_ref[...] = jnp.where(lane == 120, n_active, eid).astype(jnp.int32)


def _routing(x, router_weight):
    return pl.pallas_call(
        _routing_body,
        out_shape=(
            jax.ShapeDtypeStruct((T, E), jnp.int32),
            jax.ShapeDtypeStruct((T, 16), jnp.float32),
            jax.ShapeDtypeStruct((T, 16), jnp.float32),
            jax.ShapeDtypeStruct((1, 128), jnp.int32),
        ),
        scratch_shapes=[
            pltpu.VMEM((T, E), jnp.float32),
            pltpu.VMEM((T, E), jnp.float32),
        ],
    )(x, router_weight)


def _shared_body(x_ref, gw_ref, uw_ref, dw_ref, gl_ref, out_ref):
    x = x_ref[...]
    xb = x.astype(jnp.bfloat16)
    gw = gw_ref[...].astype(jnp.bfloat16)
    uw = uw_ref[...].astype(jnp.bfloat16)
    dw = dw_ref[...].astype(jnp.bfloat16)
    g = lax.dot_general(xb, gw, (((1,), (1,)), ((), ())),
                        preferred_element_type=jnp.float32)
    u = lax.dot_general(xb, uw, (((1,), (1,)), ((), ())),
                        preferred_element_type=jnp.float32)
    h = g * jax.nn.sigmoid(g) * u
    y = lax.dot_general(h.astype(jnp.bfloat16), dw, (((1,), (1,)), ((), ())),
                        preferred_element_type=jnp.float32)
    gate = jax.nn.sigmoid(lax.dot_general(x, gl_ref[...], (((1,), (1,)), ((), ())),
                                          preferred_element_type=jnp.float32))
    out_ref[...] = gate * y


def _shared(x, gw, uw, dw, gl):
    BT = 1024
    return pl.pallas_call(
        _shared_body,
        grid=(T // BT,),
        in_specs=[
            pl.BlockSpec((BT, D), lambda i: (i, 0)),
            pl.BlockSpec((FFS, D), lambda i: (0, 0)),
            pl.BlockSpec((FFS, D), lambda i: (0, 0)),
            pl.BlockSpec((D, FFS), lambda i: (0, 0)),
            pl.BlockSpec((1, D), lambda i: (0, 0)),
        ],
        out_specs=pl.BlockSpec((BT, D), lambda i: (i, 0)),
        out_shape=jax.ShapeDtypeStruct((T, D), jnp.float32),
    )(x, gw, uw, dw, gl)


def _dispatch(x, i0, i1):
    mesh = plsc.VectorSubcoreMesh(core_axis_name="c", subcore_axis_name="s", num_cores=2, num_subcores=16)

    @functools.partial(
        pl.kernel,
        mesh=mesh,
        out_type=jax.ShapeDtypeStruct((ROWS, D), jnp.float32),
        scratch_types=[
            pltpu.VMEM((CHT, D), jnp.float32),
            pltpu.VMEM((CHT, D), jnp.float32),
            pltpu.VMEM((CHT,), jnp.int32),
            pltpu.VMEM((CHT,), jnp.int32),
            pltpu.VMEM((CHT,), jnp.int32),
            pltpu.VMEM((CHT,), jnp.int32),
            pltpu.SemaphoreType.DMA,
            pltpu.SemaphoreType.DMA,
            pltpu.SemaphoreType.DMA,
            pltpu.SemaphoreType.DMA,
        ],
    )
    def body(x_hbm, i0_hbm, i1_hbm, xs_hbm, rows_v0, rows_v1,
             idx0_v0, idx0_v1, idx1_v0, idx1_v1, semL0, semL1, semS0, semS1):
        nc = 2
        wid = lax.axis_index("s") * nc + lax.axis_index("c")
        rows = (rows_v0, rows_v1)
        idx0 = (idx0_v0, idx0_v1)
        idx1 = (idx1_v0, idx1_v1)
        semL = (semL0, semL1)
        semS = (semS0, semS1)

        def issue_load(c):
            b = c & 1
            base = wid * TPW + c * CHT
            return pltpu.async_copy(x_hbm.at[pl.ds(base, CHT)], rows[b], semL[b])

        ldp = issue_load(0)
        scp = None
        for c in range(NCH):
            b = c & 1
            ldp.wait()
            pltpu.sync_copy(i0_hbm.at[wid, c], idx0[b])
            pltpu.sync_copy(i1_hbm.at[wid, c], idx1[b])
            if scp is not None:
                scp[0].wait()
                scp[1].wait()
            if c + 1 < NCH:
                ldp = issue_load(c + 1)
            scp = (pltpu.async_copy(rows[b], xs_hbm.at[idx0[b]], semS[b]),
                   pltpu.async_copy(rows[b], xs_hbm.at[idx1[b]], semS[b]))
        scp[0].wait()
        scp[1].wait()

    return body(x, i0, i1)


def _grouped_body(sp_ref, xs_ref, gw_hbm, uw_hbm, dw_hbm, out_ref,
                  gw_v, uw_v, dw_v, st_ref, sem_g, sem_u, sem_d):
    i = pl.program_id(0)
    n_active = sp_ref[120]
    e = sp_ref[i]

    @pl.when(i == 0)
    def _():
        pltpu.make_async_copy(gw_hbm.at[e], gw_v.at[0], sem_g).start()
        pltpu.make_async_copy(uw_hbm.at[e], uw_v.at[0], sem_u).start()
        pltpu.make_async_copy(dw_hbm.at[e], dw_v.at[0], sem_d).start()
        pltpu.make_async_copy(gw_hbm.at[e], gw_v.at[0], sem_g).wait()
        pltpu.make_async_copy(uw_hbm.at[e], uw_v.at[0], sem_u).wait()
        pltpu.make_async_copy(dw_hbm.at[e], dw_v.at[0], sem_d).wait()
        st_ref[0] = e
        st_ref[1] = 0
        st_ref[2] = 0

    @pl.when((i > 0) & (e != st_ref[0]))
    def _():
        slot = 1 - st_ref[1]
        pltpu.make_async_copy(gw_hbm.at[e], gw_v.at[slot], sem_g).wait()
        pltpu.make_async_copy(uw_hbm.at[e], uw_v.at[slot], sem_u).wait()
        pltpu.make_async_copy(dw_hbm.at[e], dw_v.at[slot], sem_d).wait()
        st_ref[0] = e
        st_ref[1] = slot
        st_ref[2] = 0

    nxt = sp_ref[jnp.minimum(i + 1, NT - 1)]

    @pl.when((st_ref[2] == 0) & (nxt != st_ref[0]) & (i < NT - 1))
    def _():
        slot = 1 - st_ref[1]
        pltpu.make_async_copy(gw_hbm.at[nxt], gw_v.at[slot], sem_g).start()
        pltpu.make_async_copy(uw_hbm.at[nxt], uw_v.at[slot], sem_u).start()
        pltpu.make_async_copy(dw_hbm.at[nxt], dw_v.at[slot], sem_d).start()
        st_ref[2] = 1

    @pl.when(i < n_active)
    def _():
        slot = st_ref[1]
        xb = xs_ref[...].astype(jnp.bfloat16)
        gwe = gw_v[pl.ds(slot, 1)][0].astype(jnp.bfloat16)
        uwe = uw_v[pl.ds(slot, 1)][0].astype(jnp.bfloat16)
        dwe = dw_v[pl.ds(slot, 1)][0].astype(jnp.bfloat16)
        g = lax.dot_general(xb, gwe, (((1,), (1,)), ((), ())),
                            preferred_element_type=jnp.float32)
        u = lax.dot_general(xb, uwe, (((1,), (1,)), ((), ())),
                            preferred_element_type=jnp.float32)
        h = g * jax.nn.sigmoid(g) * u
        out_ref[...] = lax.dot_general(h.astype(jnp.bfloat16), dwe,
                                       (((1,), (1,)), ((), ())),
                                       preferred_element_type=jnp.float32)


def _grouped(meta128, xs, gw, uw, dw):
    grid_spec = pltpu.PrefetchScalarGridSpec(
        num_scalar_prefetch=1,
        grid=(NT,),
        in_specs=[
            pl.BlockSpec((TILE_M, D), lambda i, sp: (i, 0)),
            pl.BlockSpec(memory_space=pl.ANY),
            pl.BlockSpec(memory_space=pl.ANY),
            pl.BlockSpec(memory_space=pl.ANY),
        ],
        out_specs=pl.BlockSpec((TILE_M, D), lambda i, sp: (i, 0)),
        scratch_shapes=[
            pltpu.VMEM((2, FF, D), jnp.float32),
            pltpu.VMEM((2, FF, D), jnp.float32),
            pltpu.VMEM((2, D, FF), jnp.float32),
            pltpu.SMEM((3,), jnp.int32),
            pltpu.SemaphoreType.DMA,
            pltpu.SemaphoreType.DMA,
            pltpu.SemaphoreType.DMA,
        ],
    )
    return pl.pallas_call(
        _grouped_body,
        grid_spec=grid_spec,
        out_shape=jax.ShapeDtypeStruct((ROWS, D), jnp.float32),
    )(meta128, xs, gw, uw, dw)


def _combine(ys, shared_pre, i0, i1, wb0, wb1):
    mesh = plsc.VectorSubcoreMesh(core_axis_name="c", subcore_axis_name="s", num_cores=2, num_subcores=16)

    @functools.partial(
        pl.kernel,
        mesh=mesh,
        out_type=jax.ShapeDtypeStruct((T, D), jnp.float32),
        scratch_types=[
            pltpu.VMEM((CHC, D), jnp.float32),
            pltpu.VMEM((CHC, D), jnp.float32),
            pltpu.VMEM((CHC, D), jnp.float32),
            pltpu.VMEM((CHC, D), jnp.float32),
            pltpu.VMEM((CHC, D), jnp.float32),
            pltpu.VMEM((CHC, D), jnp.float32),
            pltpu.VMEM((CHC,), jnp.int32),
            pltpu.VMEM((CHC,), jnp.int32),
            pltpu.VMEM((CHC,), jnp.int32),
            pltpu.VMEM((CHC,), jnp.int32),
            pltpu.VMEM((CHC, 16), jnp.float32),
            pltpu.VMEM((CHC, 16), jnp.float32),
            pltpu.VMEM((CHC, 16), jnp.float32),
            pltpu.VMEM((CHC, 16), jnp.float32),
            pltpu.SemaphoreType.DMA,
            pltpu.SemaphoreType.DMA,
        ],
    )
    def body(ys_hbm, sp_hbm, i0_hbm, i1_hbm, wb0_hbm, wb1_hbm, out_hbm,
             acc_v0, acc_v1, r0_v0, r0_v1, r1_v0, r1_v1,
             idx0_v0, idx0_v1, idx1_v0, idx1_v1,
             w0_v0, w0_v1, w1_v0, w1_v1, sem0, sem1):
        nc = 2
        wid = lax.axis_index("s") * nc + lax.axis_index("c")
        acc = (acc_v0, acc_v1)
        r0 = (r0_v0, r0_v1)
        r1 = (r1_v0, r1_v1)
        idx0 = (idx0_v0, idx0_v1)
        idx1 = (idx1_v0, idx1_v1)
        w0 = (w0_v0, w0_v1)
        w1 = (w1_v0, w1_v1)
        sems = (sem0, sem1)

        def issue(c):
            b = c & 1
            base = wid * TPW + c * CHC
            pltpu.sync_copy(i0_hbm.at[wid, c], idx0[b])
            pltpu.sync_copy(i1_hbm.at[wid, c], idx1[b])
            pltpu.sync_copy(wb0_hbm.at[wid, c], w0[b])
            pltpu.sync_copy(wb1_hbm.at[wid, c], w1[b])
            return (pltpu.async_copy(sp_hbm.at[pl.ds(base, CHC)], acc[b], sems[b]),
                    pltpu.async_copy(ys_hbm.at[idx0[b]], r0[b], sems[b]),
                    pltpu.async_copy(ys_hbm.at[idx1[b]], r1[b], sems[b]))

        pend = issue(0)
        for c in range(NCC):
            b = c & 1
            nxt = issue(c + 1) if c + 1 < NCC else None
            for cp in pend:
                cp.wait()

            for r in range(CHC):
                w0r = w0[b][r, :]
                w1r = w1[b][r, :]

                def vstep(j, _, w0r=w0r, w1r=w1r, r=r):
                    sl = pl.ds(j * 16, 16)
                    acc[b][r, sl] = (acc[b][r, sl] + w0r * r0[b][r, sl]
                                     + w1r * r1[b][r, sl])
                    return 0

                lax.fori_loop(0, D // 16, vstep, 0)
            base = wid * TPW + c * CHC
            pltpu.sync_copy(acc[b], out_hbm.at[pl.ds(base, CHC)])
            pend = nxt

    return body(ys, shared_pre, i0, i1, wb0, wb1)


def kernel(hidden_states, router_weight, expert_gate_w, expert_up_w,
           expert_down_w, shared_gate_w, shared_up_w, shared_down_w,
           shared_gate_lin_w):
    b, s, d = hidden_states.shape
    x = hidden_states.reshape(T, D)

    dst, wb0, wb1, meta = _routing(x, router_weight)

    i0 = dst[:, 0].reshape(NW, NCH, CHT)
    i1 = dst[:, 1].reshape(NW, NCH, CHT)
    i0c = dst[:, 0].reshape(NW, NCC, CHC)
    i1c = dst[:, 1].reshape(NW, NCC, CHC)
    wb0r = wb0.reshape(NW, NCC, CHC, 16)
    wb1r = wb1.reshape(NW, NCC, CHC, 16)
    meta128 = meta.reshape(128)

    xs = _dispatch(x, i0, i1)
    shared_pre = _shared(x, shared_gate_w, shared_up_w, shared_down_w,
                         shared_gate_lin_w)
    ys = _grouped(meta128, xs, expert_gate_w, expert_up_w, expert_down_w)
    out = _combine(ys, shared_pre, i0c, i1c, wb0r, wb1r)
    return out.reshape(b, s, d)


# speedup vs baseline: 1.2924x; 1.2924x over previous
import functools

import jax
import jax.numpy as jnp
from jax import lax
from jax.experimental import pallas as pl
from jax.experimental.pallas import tpu as pltpu
from jax.experimental.pallas import tpu_sc as plsc

E = 8
TOPK = 2
D = 2048
FF = 512
FFS = 512
T = 2048
TILE_M = 512
NT = 16
ROWS = NT * TILE_M
NW = 32
TPW = T // NW
CHT = 16
NCH = TPW // CHT
CHC = 8
NCC = TPW // CHC


def _routing_body(x_ref, rw_ref, dst_ref, wb0_ref, wb1_ref, meta_ref, contrib_ref, cum_ref):
    x = x_ref[...]
    rw = rw_ref[...]
    logits = lax.dot_general(x, rw, (((1,), (1,)), ((), ())),
                             preferred_element_type=jnp.float32)
    m = jnp.max(logits, axis=1, keepdims=True)
    ex = jnp.exp(logits - m)
    probs = ex / jnp.sum(ex, axis=1, keepdims=True)

    iota8 = lax.broadcasted_iota(jnp.int32, (T, E), 1)
    v1 = jnp.max(probs, axis=1, keepdims=True)
    i1 = jnp.min(jnp.where(probs == v1, iota8, E), axis=1, keepdims=True)
    p2 = jnp.where(iota8 == i1, -1.0, probs)
    v2 = jnp.max(p2, axis=1, keepdims=True)
    i2 = jnp.min(jnp.where(p2 == v2, iota8, E), axis=1, keepdims=True)
    denom = v1 + v2
    w1 = v1 / denom
    w2 = v2 / denom

    oh1 = (iota8 == i1).astype(jnp.float32)
    oh2 = (iota8 == i2).astype(jnp.float32)
    contrib_ref[...] = oh1 + oh2

    CH = 256
    r = lax.broadcasted_iota(jnp.int32, (CH, CH), 0)
    c = lax.broadcasted_iota(jnp.int32, (CH, CH), 1)
    tri = (r > c).astype(jnp.float32)

    def step(ch, carry):
        blk = contrib_ref[pl.ds(ch * CH, CH), :]
        cum_blk = lax.dot_general(tri, blk, (((1,), (0,)), ((), ())),
                                  preferred_element_type=jnp.float32)
        cum_ref[pl.ds(ch * CH, CH), :] = cum_blk + carry
        return carry + jnp.sum(blk, axis=0, keepdims=True)

    counts = lax.fori_loop(0, T // CH, step, jnp.zeros((1, E), jnp.float32))

    ci = counts.astype(jnp.int32)
    ntiles = (ci + (TILE_M - 1)) // TILE_M
    e_r = lax.broadcasted_iota(jnp.int32, (E, E), 0)
    e_c = lax.broadcasted_iota(jnp.int32, (E, E), 1)
    mlt = (e_r < e_c).astype(jnp.float32)
    tile_off = lax.dot_general(ntiles.astype(jnp.float32), mlt,
                               (((1,), (0,)), ((), ())),
                               preferred_element_type=jnp.float32)
    row_off = tile_off * float(TILE_M)

    cum = cum_ref[...]
    sel1 = (iota8 == i1).astype(jnp.float32)
    sel2 = (iota8 == i2).astype(jnp.float32)
    dst0 = jnp.sum((cum + row_off) * sel1, axis=1, keepdims=True)
    dst1 = jnp.sum((cum + row_off) * sel2, axis=1, keepdims=True)

    colsel0 = (iota8 == 0).astype(jnp.float32)
    colsel1 = (iota8 == 1).astype(jnp.float32)
    dst_ref[...] = (dst0 * colsel0 + dst1 * colsel1).astype(jnp.int32)
    wb0_ref[...] = jnp.broadcast_to(w1, (T, 16))
    wb1_ref[...] = jnp.broadcast_to(w2, (T, 16))

    tile_end = (tile_off + ntiles.astype(jnp.float32))
    i8 = (e_r == e_c).astype(jnp.float32)
    ends_col = lax.dot_general(i8, tile_end, (((1,), (1,)), ((), ())),
                               preferred_element_type=jnp.float32)
    ends_b = jnp.broadcast_to(ends_col, (E, 128))
    jot = lax.broadcasted_iota(jnp.int32, (E, 128), 1).astype(jnp.float32)
    eid = jnp.sum((ends_b <= jot).astype(jnp.float32), axis=0, keepdims=True)
    eid = jnp.minimum(eid, float(E - 1))
    n_active = jnp.sum(ntiles.astype(jnp.float32), axis=1, keepdims=True)
    lane = lax.broadcasted_iota(jnp.int32, (1, 128), 1)
    meta_ref[...] = jnp.where(lane == 120, n_active, eid).astype(jnp.int32)


def _routing(x, router_weight):
    return pl.pallas_call(
        _routing_body,
        out_shape=(
            jax.ShapeDtypeStruct((T, E), jnp.int32),
            jax.ShapeDtypeStruct((T, 16), jnp.float32),
            jax.ShapeDtypeStruct((T, 16), jnp.float32),
            jax.ShapeDtypeStruct((1, 128), jnp.int32),
        ),
        scratch_shapes=[
            pltpu.VMEM((T, E), jnp.float32),
            pltpu.VMEM((T, E), jnp.float32),
        ],
    )(x, router_weight)


def _shared_body(x_ref, gw_ref, uw_ref, dw_ref, gl_ref, out_ref):
    x = x_ref[...]
    xb = x.astype(jnp.bfloat16)
    gw = gw_ref[...].astype(jnp.bfloat16)
    uw = uw_ref[...].astype(jnp.bfloat16)
    dw = dw_ref[...].astype(jnp.bfloat16)
    g = lax.dot_general(xb, gw, (((1,), (1,)), ((), ())),
                        preferred_element_type=jnp.float32)
    u = lax.dot_general(xb, uw, (((1,), (1,)), ((), ())),
                        preferred_element_type=jnp.float32)
    h = g * jax.nn.sigmoid(g) * u
    y = lax.dot_general(h.astype(jnp.bfloat16), dw, (((1,), (1,)), ((), ())),
                        preferred_element_type=jnp.float32)
    gate = jax.nn.sigmoid(lax.dot_general(x, gl_ref[...], (((1,), (1,)), ((), ())),
                                          preferred_element_type=jnp.float32))
    out_ref[...] = gate * y


def _shared(x, gw, uw, dw, gl):
    BT = 1024
    return pl.pallas_call(
        _shared_body,
        grid=(T // BT,),
        in_specs=[
            pl.BlockSpec((BT, D), lambda i: (i, 0)),
            pl.BlockSpec((FFS, D), lambda i: (0, 0)),
            pl.BlockSpec((FFS, D), lambda i: (0, 0)),
            pl.BlockSpec((D, FFS), lambda i: (0, 0)),
            pl.BlockSpec((1, D), lambda i: (0, 0)),
        ],
        out_specs=pl.BlockSpec((BT, D), lambda i: (i, 0)),
        out_shape=jax.ShapeDtypeStruct((T, D), jnp.float32),
    )(x, gw, uw, dw, gl)


def _dispatch(x, i0, i1):
    mesh = plsc.VectorSubcoreMesh(core_axis_name="c", subcore_axis_name="s", num_cores=2, num_subcores=16)

    @functools.partial(
        pl.kernel,
        mesh=mesh,
        out_type=jax.ShapeDtypeStruct((ROWS, D), jnp.float32),
        scratch_types=[
            pltpu.VMEM((CHT, D), jnp.float32),
            pltpu.VMEM((CHT, D), jnp.float32),
            pltpu.VMEM((CHT,), jnp.int32),
            pltpu.VMEM((CHT,), jnp.int32),
            pltpu.VMEM((CHT,), jnp.int32),
            pltpu.VMEM((CHT,), jnp.int32),
            pltpu.SemaphoreType.DMA,
            pltpu.SemaphoreType.DMA,
            pltpu.SemaphoreType.DMA,
            pltpu.SemaphoreType.DMA,
        ],
    )
    def body(x_hbm, i0_hbm, i1_hbm, xs_hbm, rows_v0, rows_v1,
             idx0_v0, idx0_v1, idx1_v0, idx1_v1, semL0, semL1, semS0, semS1):
        nc = 2
        wid = lax.axis_index("s") * nc + lax.axis_index("c")
        rows = (rows_v0, rows_v1)
        idx0 = (idx0_v0, idx0_v1)
        idx1 = (idx1_v0, idx1_v1)
        semL = (semL0, semL1)
        semS = (semS0, semS1)

        def issue_load(c):
            b = c & 1
            base = wid * TPW + c * CHT
            return pltpu.async_copy(x_hbm.at[pl.ds(base, CHT)], rows[b], semL[b])

        ldp = issue_load(0)
        scp = None
        for c in range(NCH):
            b = c & 1
            ldp.wait()
            pltpu.sync_copy(i0_hbm.at[wid, c], idx0[b])
            pltpu.sync_copy(i1_hbm.at[wid, c], idx1[b])
            if scp is not None:
                scp[0].wait()
                scp[1].wait()
            if c + 1 < NCH:
                ldp = issue_load(c + 1)
            scp = (pltpu.async_copy(rows[b], xs_hbm.at[idx0[b]], semS[b]),
                   pltpu.async_copy(rows[b], xs_hbm.at[idx1[b]], semS[b]))
        scp[0].wait()
        scp[1].wait()

    return body(x, i0, i1)


def _grouped_body(sp_ref, xs_ref, gw_ref, uw_ref, dw_ref, out_ref):
    i = pl.program_id(0)
    n_active = sp_ref[120]

    @pl.when(i < n_active)
    def _():
        xb = xs_ref[...].astype(jnp.bfloat16)
        gwe = gw_ref[0].astype(jnp.bfloat16)
        uwe = uw_ref[0].astype(jnp.bfloat16)
        dwe = dw_ref[0].astype(jnp.bfloat16)
        g = lax.dot_general(xb, gwe, (((1,), (1,)), ((), ())),
                            preferred_element_type=jnp.float32)
        u = lax.dot_general(xb, uwe, (((1,), (1,)), ((), ())),
                            preferred_element_type=jnp.float32)
        h = g * jax.nn.sigmoid(g) * u
        out_ref[...] = lax.dot_general(h.astype(jnp.bfloat16), dwe,
                                       (((1,), (1,)), ((), ())),
                                       preferred_element_type=jnp.float32)


def _grouped(meta128, xs, gw, uw, dw):
    grid_spec = pltpu.PrefetchScalarGridSpec(
        num_scalar_prefetch=1,
        grid=(NT,),
        in_specs=[
            pl.BlockSpec((TILE_M, D), lambda i, sp: (i, 0)),
            pl.BlockSpec((1, FF, D), lambda i, sp: (sp[i], 0, 0)),
            pl.BlockSpec((1, FF, D), lambda i, sp: (sp[i], 0, 0)),
            pl.BlockSpec((1, D, FF), lambda i, sp: (sp[i], 0, 0)),
        ],
        out_specs=pl.BlockSpec((TILE_M, D), lambda i, sp: (i, 0)),
    )
    return pl.pallas_call(
        _grouped_body,
        grid_spec=grid_spec,
        out_shape=jax.ShapeDtypeStruct((ROWS, D), jnp.float32),
    )(meta128, xs, gw, uw, dw)


def _combine(ys, shared_pre, i0, i1, wb0, wb1):
    mesh = plsc.VectorSubcoreMesh(core_axis_name="c", subcore_axis_name="s", num_cores=2, num_subcores=16)

    @functools.partial(
        pl.kernel,
        mesh=mesh,
        out_type=jax.ShapeDtypeStruct((T, D), jnp.float32),
        scratch_types=[
            pltpu.VMEM((CHC, D), jnp.float32),
            pltpu.VMEM((CHC, D), jnp.float32),
            pltpu.VMEM((CHC, D), jnp.float32),
            pltpu.VMEM((CHC, D), jnp.float32),
            pltpu.VMEM((CHC, D), jnp.float32),
            pltpu.VMEM((CHC, D), jnp.float32),
            pltpu.VMEM((CHC,), jnp.int32),
            pltpu.VMEM((CHC,), jnp.int32),
            pltpu.VMEM((CHC,), jnp.int32),
            pltpu.VMEM((CHC,), jnp.int32),
            pltpu.VMEM((CHC, 16), jnp.float32),
            pltpu.VMEM((CHC, 16), jnp.float32),
            pltpu.VMEM((CHC, 16), jnp.float32),
            pltpu.VMEM((CHC, 16), jnp.float32),
            pltpu.SemaphoreType.DMA,
            pltpu.SemaphoreType.DMA,
        ],
    )
    def body(ys_hbm, sp_hbm, i0_hbm, i1_hbm, wb0_hbm, wb1_hbm, out_hbm,
             acc_v0, acc_v1, r0_v0, r0_v1, r1_v0, r1_v1,
             idx0_v0, idx0_v1, idx1_v0, idx1_v1,
             w0_v0, w0_v1, w1_v0, w1_v1, sem0, sem1):
        nc = 2
        wid = lax.axis_index("s") * nc + lax.axis_index("c")
        acc = (acc_v0, acc_v1)
        r0 = (r0_v0, r0_v1)
        r1 = (r1_v0, r1_v1)
        idx0 = (idx0_v0, idx0_v1)
        idx1 = (idx1_v0, idx1_v1)
        w0 = (w0_v0, w0_v1)
        w1 = (w1_v0, w1_v1)
        sems = (sem0, sem1)

        def issue(c):
            b = c & 1
            base = wid * TPW + c * CHC
            pltpu.sync_copy(i0_hbm.at[wid, c], idx0[b])
            pltpu.sync_copy(i1_hbm.at[wid, c], idx1[b])
            pltpu.sync_copy(wb0_hbm.at[wid, c], w0[b])
            pltpu.sync_copy(wb1_hbm.at[wid, c], w1[b])
            return (pltpu.async_copy(sp_hbm.at[pl.ds(base, CHC)], acc[b], sems[b]),
                    pltpu.async_copy(ys_hbm.at[idx0[b]], r0[b], sems[b]),
                    pltpu.async_copy(ys_hbm.at[idx1[b]], r1[b], sems[b]))

        pend = issue(0)
        for c in range(NCC):
            b = c & 1
            nxt = issue(c + 1) if c + 1 < NCC else None
            for cp in pend:
                cp.wait()

            w0v = [w0[b][r, :] for r in range(CHC)]
            w1v = [w1[b][r, :] for r in range(CHC)]

            def vstep(j, _, w0v=w0v, w1v=w1v, b=b):
                sl = pl.ds(j * 16, 16)
                for r in range(CHC):
                    acc[b][r, sl] = (acc[b][r, sl] + w0v[r] * r0[b][r, sl]
                                     + w1v[r] * r1[b][r, sl])
                return 0

            lax.fori_loop(0, D // 16, vstep, 0)
            base = wid * TPW + c * CHC
            pltpu.sync_copy(acc[b], out_hbm.at[pl.ds(base, CHC)])
            pend = nxt

    return body(ys, shared_pre, i0, i1, wb0, wb1)


def kernel(hidden_states, router_weight, expert_gate_w, expert_up_w,
           expert_down_w, shared_gate_w, shared_up_w, shared_down_w,
           shared_gate_lin_w):
    b, s, d = hidden_states.shape
    x = hidden_states.reshape(T, D)

    dst, wb0, wb1, meta = _routing(x, router_weight)

    i0 = dst[:, 0].reshape(NW, NCH, CHT)
    i1 = dst[:, 1].reshape(NW, NCH, CHT)
    i0c = dst[:, 0].reshape(NW, NCC, CHC)
    i1c = dst[:, 1].reshape(NW, NCC, CHC)
    wb0r = wb0.reshape(NW, NCC, CHC, 16)
    wb1r = wb1.reshape(NW, NCC, CHC, 16)
    meta128 = meta.reshape(128)

    xs = _dispatch(x, i0, i1)
    shared_pre = _shared(x, shared_gate_w, shared_up_w, shared_down_w,
                         shared_gate_lin_w)
    ys = _grouped(meta128, xs, expert_gate_w, expert_up_w, expert_down_w)
    out = _combine(ys, shared_pre, i0c, i1c, wb0r, wb1r)
    return out.reshape(b, s, d)
